# Initial kernel scaffold; baseline (speedup 1.0000x reference)
import jax
import jax.numpy as jnp
from jax.experimental import pallas as pl


def kernel(x, neigh_orders, W1, b1, gamma1, beta1, W2, b2, gamma2, beta2):
    raise NotImplementedError("write your pallas kernel here")


# speedup vs baseline: 1.7677x; 1.7677x over previous
import functools

import jax
import jax.numpy as jnp
from jax import lax
from jax.experimental import pallas as pl
from jax.experimental.pallas import tpu as pltpu
from jax.experimental.pallas import tpu_sc as plsc

N = 50000
C = 128
K = 7
NK = N * K

NW = 32
CHUNK = 112
BPAD = 351232
B_PER_W = BPAD // NW
NCHUNK = B_PER_W // CHUNK
NP = BPAD // K
MM_R = 512
MM_GRID = NP // MM_R


def _leaky(v):
    return jnp.where(v > 0, v, 0.2 * v)


def _make_sc_gather(table_rows):
    mesh = plsc.VectorSubcoreMesh(core_axis_name="c", subcore_axis_name="s")

    @functools.partial(
        pl.kernel,
        mesh=mesh,
        out_type=jax.ShapeDtypeStruct((BPAD, C), jnp.float32),
        scratch_types=[
            pltpu.VMEM((B_PER_W,), jnp.int32),
            pltpu.VMEM((CHUNK, C), jnp.float32),
            pltpu.SemaphoreType.DMA,
        ],
    )
    def gather_kernel(table_hbm, idx_hbm, out_hbm, idx_v, rows_v, sem):
        wid = lax.axis_index("s") * 2 + lax.axis_index("c")
        row0 = wid * B_PER_W
        pltpu.sync_copy(idx_hbm.at[pl.ds(row0, B_PER_W)], idx_v)

        def step(j, carry):
            idx_c = idx_v.at[pl.ds(j * CHUNK, CHUNK)]
            pltpu.async_copy(table_hbm.at[idx_c], rows_v, sem).wait()
            pltpu.sync_copy(rows_v, out_hbm.at[pl.ds(row0 + j * CHUNK, CHUNK)])
            return carry

        lax.fori_loop(0, NCHUNK, step, 0)

    return gather_kernel


def _mm_body(g_ref, w_ref, b_ref, sc_ref, sh_ref, h_ref, sum_ref, sq_ref,
             *, apply_affine):
    i = pl.program_id(0)
    g = g_ref[...]
    if apply_affine:
        g = _leaky(g * sc_ref[...] + sh_ref[...])
    h = jnp.dot(g, w_ref[...], preferred_element_type=jnp.float32) + b_ref[...]
    h_ref[...] = h
    rows = i * MM_R + lax.broadcasted_iota(jnp.int32, (MM_R, 1), 0)
    hm = jnp.where(rows < N, h, 0.0)
    s = jnp.sum(hm, axis=0, keepdims=True)
    q = jnp.sum(hm * hm, axis=0, keepdims=True)

    @pl.when(i == 0)
    def _init():
        sum_ref[...] = s
        sq_ref[...] = q

    @pl.when(i > 0)
    def _acc():
        sum_ref[...] += s
        sq_ref[...] += q


def _conv_mm(g, w, b, scale_t, shift_t, apply_affine):
    body = functools.partial(_mm_body, apply_affine=apply_affine)
    return pl.pallas_call(
        body,
        grid=(MM_GRID,),
        in_specs=[
            pl.BlockSpec((MM_R, K * C), lambda i: (i, 0)),
            pl.BlockSpec((K * C, C), lambda i: (0, 0)),
            pl.BlockSpec((1, C), lambda i: (0, 0)),
            pl.BlockSpec((1, K * C), lambda i: (0, 0)),
            pl.BlockSpec((1, K * C), lambda i: (0, 0)),
        ],
        out_specs=[
            pl.BlockSpec((MM_R, C), lambda i: (i, 0)),
            pl.BlockSpec((1, C), lambda i: (0, 0)),
            pl.BlockSpec((1, C), lambda i: (0, 0)),
        ],
        out_shape=[
            jax.ShapeDtypeStruct((NP, C), jnp.float32),
            jax.ShapeDtypeStruct((1, C), jnp.float32),
            jax.ShapeDtypeStruct((1, C), jnp.float32),
        ],
    )(g, w, b.reshape(1, C), scale_t, shift_t)


FIN_R = 1000


def _fin_body(h_ref, x_ref, sc_ref, sh_ref, o_ref):
    o_ref[...] = _leaky(h_ref[...] * sc_ref[...] + sh_ref[...] + x_ref[...])


def _finalize(h2, x, scale2, shift2):
    return pl.pallas_call(
        _fin_body,
        grid=(N // FIN_R,),
        in_specs=[
            pl.BlockSpec((FIN_R, C), lambda i: (i, 0)),
            pl.BlockSpec((FIN_R, C), lambda i: (i, 0)),
            pl.BlockSpec((1, C), lambda i: (0, 0)),
            pl.BlockSpec((1, C), lambda i: (0, 0)),
        ],
        out_specs=pl.BlockSpec((FIN_R, C), lambda i: (i, 0)),
        out_shape=jax.ShapeDtypeStruct((N, C), jnp.float32),
    )(h2, x, scale2.reshape(1, C), shift2.reshape(1, C))


def _bn_coeffs(ssum, ssq, gamma, beta):
    mean = ssum[0] / N
    var = ssq[0] / N - mean * mean
    scale = gamma * lax.rsqrt(var + 1e-5)
    shift = beta - mean * scale
    return scale, shift


def kernel(x, neigh_orders, W1, b1, gamma1, beta1, W2, b2, gamma2, beta2):
    idx = jnp.concatenate(
        [neigh_orders, jnp.zeros((BPAD - NK,), jnp.int32)]
    )

    zeros_kc = jnp.zeros((1, K * C), jnp.float32)

    g1 = _make_sc_gather(N)(x, idx)
    g1 = g1.reshape(NP, K * C)
    h1, s1, q1 = _conv_mm(g1, W1, b1, zeros_kc, zeros_kc, apply_affine=False)
    scale1, shift1 = _bn_coeffs(s1, q1, gamma1, beta1)

    g2 = _make_sc_gather(NP)(h1, idx)
    g2 = g2.reshape(NP, K * C)
    scale1_t = jnp.tile(scale1, K).reshape(1, K * C)
    shift1_t = jnp.tile(shift1, K).reshape(1, K * C)
    h2, s2, q2 = _conv_mm(g2, W2, b2, scale1_t, shift1_t, apply_affine=True)
    scale2, shift2 = _bn_coeffs(s2, q2, gamma2, beta2)

    return _finalize(h2, x, scale2, shift2)


# speedup vs baseline: 2.0352x; 1.1514x over previous
import functools

import jax
import jax.numpy as jnp
from jax import lax
from jax.experimental import pallas as pl
from jax.experimental.pallas import tpu as pltpu
from jax.experimental.pallas import tpu_sc as plsc

N = 50000
C = 128
K = 7
NK = N * K

NW = 32
NBUF = 4
CHUNK = 112
BPAD = 351232
B_PER_W = BPAD // NW
NCHUNK = B_PER_W // CHUNK
NP = BPAD // K
MM_R = 512
MM_GRID = NP // MM_R


def _leaky(v):
    return jnp.where(v > 0, v, 0.2 * v)


def _make_sc_gather(table_rows):
    mesh = plsc.VectorSubcoreMesh(core_axis_name="c", subcore_axis_name="s")

    @functools.partial(
        pl.kernel,
        mesh=mesh,
        out_type=jax.ShapeDtypeStruct((BPAD, C), jnp.float32),
        scratch_types=[
            pltpu.VMEM((B_PER_W,), jnp.int32),
            pltpu.VMEM((NBUF, CHUNK, C), jnp.float32),
            pltpu.SemaphoreType.DMA,
            pltpu.SemaphoreType.DMA,
        ],
    )
    def gather_kernel(table_hbm, idx_hbm, out_hbm, idx_v, rows_v, semg, semw):
        wid = lax.axis_index("s") * 2 + lax.axis_index("c")
        row0 = wid * B_PER_W
        pltpu.sync_copy(idx_hbm.at[pl.ds(row0, B_PER_W)], idx_v)

        def gstart(j):
            buf = rows_v.at[lax.rem(j, NBUF)]
            idx_c = idx_v.at[pl.ds(j * CHUNK, CHUNK)]
            pltpu.make_async_copy(table_hbm.at[idx_c], buf, semg).start()

        def gwait():
            pltpu.make_async_copy(
                table_hbm.at[idx_v.at[pl.ds(0, CHUNK)]], rows_v.at[0], semg
            ).wait()

        def wstart(j):
            pltpu.make_async_copy(
                rows_v.at[lax.rem(j, NBUF)],
                out_hbm.at[pl.ds(row0 + j * CHUNK, CHUNK)],
                semw,
            ).start()

        def wwait():
            pltpu.make_async_copy(
                rows_v.at[0], out_hbm.at[pl.ds(row0, CHUNK)], semw
            ).wait()

        gstart(0)

        def step(j, carry):
            @pl.when(j >= NBUF - 1)
            def _():
                wwait()

            @pl.when(j + 1 < NCHUNK)
            def _():
                gstart(j + 1)

            gwait()
            wstart(j)
            return carry

        lax.fori_loop(0, NCHUNK, step, 0)
        for _ in range(NBUF - 1):
            wwait()

    return gather_kernel


def _mm_body(g_ref, w_ref, b_ref, sc_ref, sh_ref, h_ref, sum_ref, sq_ref,
             *, apply_affine):
    i = pl.program_id(0)
    g = g_ref[...]
    if apply_affine:
        g = _leaky(g * sc_ref[...] + sh_ref[...])
    h = jnp.dot(g, w_ref[...], preferred_element_type=jnp.float32) + b_ref[...]
    h_ref[...] = h
    rows = i * MM_R + lax.broadcasted_iota(jnp.int32, (MM_R, 1), 0)
    hm = jnp.where(rows < N, h, 0.0)
    s = jnp.sum(hm, axis=0, keepdims=True)
    q = jnp.sum(hm * hm, axis=0, keepdims=True)

    @pl.when(i == 0)
    def _init():
        sum_ref[...] = s
        sq_ref[...] = q

    @pl.when(i > 0)
    def _acc():
        sum_ref[...] += s
        sq_ref[...] += q


def _conv_mm(g, w, b, scale_t, shift_t, apply_affine):
    body = functools.partial(_mm_body, apply_affine=apply_affine)
    return pl.pallas_call(
        body,
        grid=(MM_GRID,),
        in_specs=[
            pl.BlockSpec((MM_R, K * C), lambda i: (i, 0)),
            pl.BlockSpec((K * C, C), lambda i: (0, 0)),
            pl.BlockSpec((1, C), lambda i: (0, 0)),
            pl.BlockSpec((1, K * C), lambda i: (0, 0)),
            pl.BlockSpec((1, K * C), lambda i: (0, 0)),
        ],
        out_specs=[
            pl.BlockSpec((MM_R, C), lambda i: (i, 0)),
            pl.BlockSpec((1, C), lambda i: (0, 0)),
            pl.BlockSpec((1, C), lambda i: (0, 0)),
        ],
        out_shape=[
            jax.ShapeDtypeStruct((NP, C), jnp.float32),
            jax.ShapeDtypeStruct((1, C), jnp.float32),
            jax.ShapeDtypeStruct((1, C), jnp.float32),
        ],
    )(g, w, b.reshape(1, C), scale_t, shift_t)


FIN_R = 1000


def _fin_body(h_ref, x_ref, sc_ref, sh_ref, o_ref):
    o_ref[...] = _leaky(h_ref[...] * sc_ref[...] + sh_ref[...] + x_ref[...])


def _finalize(h2, x, scale2, shift2):
    return pl.pallas_call(
        _fin_body,
        grid=(N // FIN_R,),
        in_specs=[
            pl.BlockSpec((FIN_R, C), lambda i: (i, 0)),
            pl.BlockSpec((FIN_R, C), lambda i: (i, 0)),
            pl.BlockSpec((1, C), lambda i: (0, 0)),
            pl.BlockSpec((1, C), lambda i: (0, 0)),
        ],
        out_specs=pl.BlockSpec((FIN_R, C), lambda i: (i, 0)),
        out_shape=jax.ShapeDtypeStruct((N, C), jnp.float32),
    )(h2, x, scale2.reshape(1, C), shift2.reshape(1, C))


def _bn_coeffs(ssum, ssq, gamma, beta):
    mean = ssum[0] / N
    var = ssq[0] / N - mean * mean
    scale = gamma * lax.rsqrt(var + 1e-5)
    shift = beta - mean * scale
    return scale, shift


def kernel(x, neigh_orders, W1, b1, gamma1, beta1, W2, b2, gamma2, beta2):
    idx = jnp.concatenate(
        [neigh_orders, jnp.zeros((BPAD - NK,), jnp.int32)]
    )

    zeros_kc = jnp.zeros((1, K * C), jnp.float32)

    g1 = _make_sc_gather(N)(x, idx)
    g1 = g1.reshape(NP, K * C)
    h1, s1, q1 = _conv_mm(g1, W1, b1, zeros_kc, zeros_kc, apply_affine=False)
    scale1, shift1 = _bn_coeffs(s1, q1, gamma1, beta1)

    g2 = _make_sc_gather(NP)(h1, idx)
    g2 = g2.reshape(NP, K * C)
    scale1_t = jnp.tile(scale1, K).reshape(1, K * C)
    shift1_t = jnp.tile(shift1, K).reshape(1, K * C)
    h2, s2, q2 = _conv_mm(g2, W2, b2, scale1_t, shift1_t, apply_affine=True)
    scale2, shift2 = _bn_coeffs(s2, q2, gamma2, beta2)

    return _finalize(h2, x, scale2, shift2)


# speedup vs baseline: 2.7390x; 1.3458x over previous
import functools

import jax
import jax.numpy as jnp
import numpy as np
from jax import lax
from jax.experimental import pallas as pl
from jax.experimental.pallas import tpu as pltpu
from jax.experimental.pallas import tpu_sc as plsc

N = 50000
C = 128
H = 64
K = 7
NK = N * K

NW = 32
NBUF = 4
CHUNK = 112
BPAD = 351232
B_PER_W = BPAD // NW
NCHUNK = B_PER_W // CHUNK
NP = BPAD // K
MM_R = 512
MM_GRID = NP // MM_R

_HI_MASK = np.int32(-65536)


def _leaky(v):
    return jnp.where(v > 0, v, 0.2 * v)


def _unpack(w):
    lo = lax.bitcast_convert_type(lax.shift_left(w, 16), jnp.float32)
    hi = lax.bitcast_convert_type(lax.bitwise_and(w, _HI_MASK), jnp.float32)
    return lo, hi


def _pack(a, b):
    ar = a.astype(jnp.bfloat16).astype(jnp.float32)
    br = b.astype(jnp.bfloat16).astype(jnp.float32)
    au = lax.shift_right_logical(lax.bitcast_convert_type(ar, jnp.int32), 16)
    bu = lax.bitwise_and(lax.bitcast_convert_type(br, jnp.int32), _HI_MASK)
    return lax.bitwise_or(au, bu)


def _make_sc_gather():
    mesh = plsc.VectorSubcoreMesh(core_axis_name="c", subcore_axis_name="s")

    @functools.partial(
        pl.kernel,
        mesh=mesh,
        compiler_params=pltpu.CompilerParams(use_tc_tiling_on_sc=False),
        out_type=jax.ShapeDtypeStruct((BPAD, H), jnp.int32),
        scratch_types=[
            pltpu.VMEM((B_PER_W,), jnp.int32),
            pltpu.VMEM((NBUF, CHUNK, H), jnp.int32),
            pltpu.SemaphoreType.DMA,
            pltpu.SemaphoreType.DMA,
        ],
    )
    def gather_kernel(table_hbm, idx_hbm, out_hbm, idx_v, rows_v, semg, semw):
        wid = lax.axis_index("s") * 2 + lax.axis_index("c")
        row0 = wid * B_PER_W
        pltpu.sync_copy(idx_hbm.at[pl.ds(row0, B_PER_W)], idx_v)

        def gstart(j):
            buf = rows_v.at[lax.rem(j, NBUF)]
            idx_c = idx_v.at[pl.ds(j * CHUNK, CHUNK)]
            pltpu.make_async_copy(table_hbm.at[idx_c], buf, semg).start()

        def gwait():
            pltpu.make_async_copy(
                table_hbm.at[idx_v.at[pl.ds(0, CHUNK)]], rows_v.at[0], semg
            ).wait()

        def wstart(j):
            pltpu.make_async_copy(
                rows_v.at[lax.rem(j, NBUF)],
                out_hbm.at[pl.ds(row0 + j * CHUNK, CHUNK)],
                semw,
            ).start()

        def wwait():
            pltpu.make_async_copy(
                rows_v.at[0], out_hbm.at[pl.ds(row0, CHUNK)], semw
            ).wait()

        gstart(0)

        def step(j, carry):
            @pl.when(j >= NBUF - 1)
            def _():
                wwait()

            @pl.when(j + 1 < NCHUNK)
            def _():
                gstart(j + 1)

            gwait()
            wstart(j)
            return carry

        lax.fori_loop(0, NCHUNK, step, 0)
        for _ in range(NBUF - 1):
            wwait()

    return gather_kernel


PACK_R = 1000


def _pack_x_body(x_ref, o_ref):
    o_ref[...] = _pack(x_ref[:, :H], x_ref[:, H:])


def _pack_x(x):
    return pl.pallas_call(
        _pack_x_body,
        grid=(N // PACK_R,),
        in_specs=[pl.BlockSpec((PACK_R, C), lambda i: (i, 0))],
        out_specs=pl.BlockSpec((PACK_R, H), lambda i: (i, 0)),
        out_shape=jax.ShapeDtypeStruct((N, H), jnp.int32),
    )(x)


def _mm_body(g_ref, wlo_ref, whi_ref, b_ref, sclo_ref, schi_ref, shlo_ref,
             shhi_ref, h_ref, sum_ref, sq_ref, *, apply_affine):
    i = pl.program_id(0)
    glo, ghi = _unpack(g_ref[...])
    if apply_affine:
        glo = _leaky(glo * sclo_ref[...] + shlo_ref[...])
        ghi = _leaky(ghi * schi_ref[...] + shhi_ref[...])
    h = (
        jnp.dot(glo, wlo_ref[...], preferred_element_type=jnp.float32)
        + jnp.dot(ghi, whi_ref[...], preferred_element_type=jnp.float32)
        + b_ref[...]
    )
    h_ref[...] = _pack(h[:, :H], h[:, H:])
    rows = i * MM_R + lax.broadcasted_iota(jnp.int32, (MM_R, 1), 0)
    hm = jnp.where(rows < N, h, 0.0)
    s = jnp.sum(hm, axis=0, keepdims=True)
    q = jnp.sum(hm * hm, axis=0, keepdims=True)

    @pl.when(i == 0)
    def _init():
        sum_ref[...] = s
        sq_ref[...] = q

    @pl.when(i > 0)
    def _acc():
        sum_ref[...] += s
        sq_ref[...] += q


def _conv_mm(g, wlo, whi, b, sclo, schi, shlo, shhi, apply_affine):
    body = functools.partial(_mm_body, apply_affine=apply_affine)
    kh = K * H
    return pl.pallas_call(
        body,
        grid=(MM_GRID,),
        in_specs=[
            pl.BlockSpec((MM_R, kh), lambda i: (i, 0)),
            pl.BlockSpec((kh, C), lambda i: (0, 0)),
            pl.BlockSpec((kh, C), lambda i: (0, 0)),
            pl.BlockSpec((1, C), lambda i: (0, 0)),
            pl.BlockSpec((1, kh), lambda i: (0, 0)),
            pl.BlockSpec((1, kh), lambda i: (0, 0)),
            pl.BlockSpec((1, kh), lambda i: (0, 0)),
            pl.BlockSpec((1, kh), lambda i: (0, 0)),
        ],
        out_specs=[
            pl.BlockSpec((MM_R, H), lambda i: (i, 0)),
            pl.BlockSpec((1, C), lambda i: (0, 0)),
            pl.BlockSpec((1, C), lambda i: (0, 0)),
        ],
        out_shape=[
            jax.ShapeDtypeStruct((NP, H), jnp.int32),
            jax.ShapeDtypeStruct((1, C), jnp.float32),
            jax.ShapeDtypeStruct((1, C), jnp.float32),
        ],
    )(g, wlo, whi, b.reshape(1, C), sclo, schi, shlo, shhi)


FIN_R = 1000


def _fin_body(h_ref, x_ref, sc_ref, sh_ref, o_ref):
    hlo, hhi = _unpack(h_ref[...])
    h = jnp.concatenate([hlo, hhi], axis=1)
    o_ref[...] = _leaky(h * sc_ref[...] + sh_ref[...] + x_ref[...])


def _finalize(h2, x, scale2, shift2):
    return pl.pallas_call(
        _fin_body,
        grid=(N // FIN_R,),
        in_specs=[
            pl.BlockSpec((FIN_R, H), lambda i: (i, 0)),
            pl.BlockSpec((FIN_R, C), lambda i: (i, 0)),
            pl.BlockSpec((1, C), lambda i: (0, 0)),
            pl.BlockSpec((1, C), lambda i: (0, 0)),
        ],
        out_specs=pl.BlockSpec((FIN_R, C), lambda i: (i, 0)),
        out_shape=jax.ShapeDtypeStruct((N, C), jnp.float32),
    )(h2, x, scale2.reshape(1, C), shift2.reshape(1, C))


def _bn_coeffs(ssum, ssq, gamma, beta):
    mean = ssum[0] / N
    var = ssq[0] / N - mean * mean
    scale = gamma * lax.rsqrt(var + 1e-5)
    shift = beta - mean * scale
    return scale, shift


def _split_w(w):
    w3 = w.reshape(K, C, C)
    wlo = w3[:, :H, :].reshape(K * H, C)
    whi = w3[:, H:, :].reshape(K * H, C)
    return wlo, whi


def kernel(x, neigh_orders, W1, b1, gamma1, beta1, W2, b2, gamma2, beta2):
    idx = jnp.concatenate(
        [neigh_orders, jnp.zeros((BPAD - NK,), jnp.int32)]
    )
    kh = K * H
    zeros_kh = jnp.zeros((1, kh), jnp.float32)
    w1lo, w1hi = _split_w(W1)
    w2lo, w2hi = _split_w(W2)

    gather = _make_sc_gather()

    xp = _pack_x(x)
    g1 = gather(xp, idx).reshape(NP, kh)
    h1, s1, q1 = _conv_mm(g1, w1lo, w1hi, b1, zeros_kh, zeros_kh, zeros_kh,
                          zeros_kh, apply_affine=False)
    scale1, shift1 = _bn_coeffs(s1, q1, gamma1, beta1)

    g2 = gather(h1, idx).reshape(NP, kh)
    sc1lo = jnp.tile(scale1[:H], K).reshape(1, kh)
    sc1hi = jnp.tile(scale1[H:], K).reshape(1, kh)
    sh1lo = jnp.tile(shift1[:H], K).reshape(1, kh)
    sh1hi = jnp.tile(shift1[H:], K).reshape(1, kh)
    h2, s2, q2 = _conv_mm(g2, w2lo, w2hi, b2, sc1lo, sc1hi, sh1lo, sh1hi,
                          apply_affine=True)
    scale2, shift2 = _bn_coeffs(s2, q2, gamma2, beta2)

    return _finalize(h2, x, scale2, shift2)


# speedup vs baseline: 2.9469x; 1.0759x over previous
import functools

import jax
import jax.numpy as jnp
import numpy as np
from jax import lax
from jax.experimental import pallas as pl
from jax.experimental.pallas import tpu as pltpu
from jax.experimental.pallas import tpu_sc as plsc

N = 50000
C = 128
H = 64
K = 7
NK = N * K

NW = 32
NBUF = 4
CHUNK = 112
BPAD = 351232
B_PER_W = BPAD // NW
NCHUNK = B_PER_W // CHUNK
NP = BPAD // K
MM_R = 1024
MM_GRID = NP // MM_R

_HI_MASK = np.int32(-65536)


def _leaky(v):
    return jnp.where(v > 0, v, 0.2 * v)


def _unpack(w):
    lo = lax.bitcast_convert_type(lax.shift_left(w, 16), jnp.float32)
    hi = lax.bitcast_convert_type(lax.bitwise_and(w, _HI_MASK), jnp.float32)
    return lo, hi


def _pack(a, b):
    ar = a.astype(jnp.bfloat16).astype(jnp.float32)
    br = b.astype(jnp.bfloat16).astype(jnp.float32)
    au = lax.shift_right_logical(lax.bitcast_convert_type(ar, jnp.int32), 16)
    bu = lax.bitwise_and(lax.bitcast_convert_type(br, jnp.int32), _HI_MASK)
    return lax.bitwise_or(au, bu)


def _make_sc_gather():
    mesh = plsc.VectorSubcoreMesh(core_axis_name="c", subcore_axis_name="s")

    @functools.partial(
        pl.kernel,
        mesh=mesh,
        compiler_params=pltpu.CompilerParams(use_tc_tiling_on_sc=False),
        out_type=jax.ShapeDtypeStruct((BPAD, H), jnp.int32),
        scratch_types=[
            pltpu.VMEM((B_PER_W,), jnp.int32),
            pltpu.VMEM((NBUF, CHUNK, H), jnp.int32),
            pltpu.SemaphoreType.DMA,
            pltpu.SemaphoreType.DMA,
        ],
    )
    def gather_kernel(table_hbm, idx_hbm, out_hbm, idx_v, rows_v, semg, semw):
        wid = lax.axis_index("s") * 2 + lax.axis_index("c")
        row0 = wid * B_PER_W
        pltpu.sync_copy(idx_hbm.at[pl.ds(row0, B_PER_W)], idx_v)

        def gstart(j):
            buf = rows_v.at[lax.rem(j, NBUF)]
            idx_c = idx_v.at[pl.ds(j * CHUNK, CHUNK)]
            pltpu.make_async_copy(table_hbm.at[idx_c], buf, semg).start()

        def gwait():
            pltpu.make_async_copy(
                table_hbm.at[idx_v.at[pl.ds(0, CHUNK)]], rows_v.at[0], semg
            ).wait()

        def wstart(j):
            pltpu.make_async_copy(
                rows_v.at[lax.rem(j, NBUF)],
                out_hbm.at[pl.ds(row0 + j * CHUNK, CHUNK)],
                semw,
            ).start()

        def wwait():
            pltpu.make_async_copy(
                rows_v.at[0], out_hbm.at[pl.ds(row0, CHUNK)], semw
            ).wait()

        gstart(0)

        def step(j, carry):
            @pl.when(j >= NBUF - 1)
            def _():
                wwait()

            @pl.when(j + 1 < NCHUNK)
            def _():
                gstart(j + 1)

            gwait()
            wstart(j)
            return carry

        lax.fori_loop(0, NCHUNK, step, 0)
        for _ in range(NBUF - 1):
            wwait()

    return gather_kernel


PACK_R = 1000


def _pack_x_body(x_ref, o_ref):
    o_ref[...] = _pack(x_ref[:, :H], x_ref[:, H:])


def _pack_x(x):
    return pl.pallas_call(
        _pack_x_body,
        grid=(N // PACK_R,),
        in_specs=[pl.BlockSpec((PACK_R, C), lambda i: (i, 0))],
        out_specs=pl.BlockSpec((PACK_R, H), lambda i: (i, 0)),
        out_shape=jax.ShapeDtypeStruct((N, H), jnp.int32),
    )(x)


def _mm_body(g_ref, wlo_ref, whi_ref, b_ref, sclo_ref, schi_ref, shlo_ref,
             shhi_ref, h_ref, sum_ref, sq_ref, *, apply_affine):
    i = pl.program_id(0)
    glo, ghi = _unpack(g_ref[...])
    if apply_affine:
        glo = _leaky(glo * sclo_ref[...] + shlo_ref[...])
        ghi = _leaky(ghi * schi_ref[...] + shhi_ref[...])
    h = (
        jnp.dot(glo.astype(jnp.bfloat16), wlo_ref[...],
                preferred_element_type=jnp.float32)
        + jnp.dot(ghi.astype(jnp.bfloat16), whi_ref[...],
                  preferred_element_type=jnp.float32)
        + b_ref[...]
    )
    h_ref[...] = _pack(h[:, :H], h[:, H:])
    rows = i * MM_R + lax.broadcasted_iota(jnp.int32, (MM_R, 1), 0)
    hm = jnp.where(rows < N, h, 0.0)
    s = jnp.sum(hm, axis=0, keepdims=True)
    q = jnp.sum(hm * hm, axis=0, keepdims=True)

    @pl.when(i == 0)
    def _init():
        sum_ref[...] = s
        sq_ref[...] = q

    @pl.when(i > 0)
    def _acc():
        sum_ref[...] += s
        sq_ref[...] += q


def _conv_mm(g, wlo, whi, b, sclo, schi, shlo, shhi, apply_affine):
    body = functools.partial(_mm_body, apply_affine=apply_affine)
    kh = K * H
    return pl.pallas_call(
        body,
        grid=(MM_GRID,),
        in_specs=[
            pl.BlockSpec((MM_R, kh), lambda i: (i, 0)),
            pl.BlockSpec((kh, C), lambda i: (0, 0)),
            pl.BlockSpec((kh, C), lambda i: (0, 0)),
            pl.BlockSpec((1, C), lambda i: (0, 0)),
            pl.BlockSpec((1, kh), lambda i: (0, 0)),
            pl.BlockSpec((1, kh), lambda i: (0, 0)),
            pl.BlockSpec((1, kh), lambda i: (0, 0)),
            pl.BlockSpec((1, kh), lambda i: (0, 0)),
        ],
        out_specs=[
            pl.BlockSpec((MM_R, H), lambda i: (i, 0)),
            pl.BlockSpec((1, C), lambda i: (0, 0)),
            pl.BlockSpec((1, C), lambda i: (0, 0)),
        ],
        out_shape=[
            jax.ShapeDtypeStruct((NP, H), jnp.int32),
            jax.ShapeDtypeStruct((1, C), jnp.float32),
            jax.ShapeDtypeStruct((1, C), jnp.float32),
        ],
    )(g, wlo, whi, b.reshape(1, C), sclo, schi, shlo, shhi)


FIN_R = 1000


def _fin_body(h_ref, x_ref, sc_ref, sh_ref, o_ref):
    hlo, hhi = _unpack(h_ref[...])
    h = jnp.concatenate([hlo, hhi], axis=1)
    o_ref[...] = _leaky(h * sc_ref[...] + sh_ref[...] + x_ref[...])


def _finalize(h2, x, scale2, shift2):
    return pl.pallas_call(
        _fin_body,
        grid=(N // FIN_R,),
        in_specs=[
            pl.BlockSpec((FIN_R, H), lambda i: (i, 0)),
            pl.BlockSpec((FIN_R, C), lambda i: (i, 0)),
            pl.BlockSpec((1, C), lambda i: (0, 0)),
            pl.BlockSpec((1, C), lambda i: (0, 0)),
        ],
        out_specs=pl.BlockSpec((FIN_R, C), lambda i: (i, 0)),
        out_shape=jax.ShapeDtypeStruct((N, C), jnp.float32),
    )(h2, x, scale2.reshape(1, C), shift2.reshape(1, C))


def _bn_coeffs(ssum, ssq, gamma, beta):
    mean = ssum[0] / N
    var = ssq[0] / N - mean * mean
    scale = gamma * lax.rsqrt(var + 1e-5)
    shift = beta - mean * scale
    return scale, shift


def _split_w(w):
    w3 = w.reshape(K, C, C)
    wlo = w3[:, :H, :].reshape(K * H, C)
    whi = w3[:, H:, :].reshape(K * H, C)
    return wlo, whi


def kernel(x, neigh_orders, W1, b1, gamma1, beta1, W2, b2, gamma2, beta2):
    idx = jnp.concatenate(
        [neigh_orders, jnp.zeros((BPAD - NK,), jnp.int32)]
    )
    kh = K * H
    zeros_kh = jnp.zeros((1, kh), jnp.float32)
    w1lo, w1hi = _split_w(W1.astype(jnp.bfloat16))
    w2lo, w2hi = _split_w(W2.astype(jnp.bfloat16))

    gather = _make_sc_gather()

    xp = _pack_x(x)
    g1 = gather(xp, idx).reshape(NP, kh)
    h1, s1, q1 = _conv_mm(g1, w1lo, w1hi, b1, zeros_kh, zeros_kh, zeros_kh,
                          zeros_kh, apply_affine=False)
    scale1, shift1 = _bn_coeffs(s1, q1, gamma1, beta1)

    g2 = gather(h1, idx).reshape(NP, kh)
    sc1lo = jnp.tile(scale1[:H], K).reshape(1, kh)
    sc1hi = jnp.tile(scale1[H:], K).reshape(1, kh)
    sh1lo = jnp.tile(shift1[:H], K).reshape(1, kh)
    sh1hi = jnp.tile(shift1[H:], K).reshape(1, kh)
    h2, s2, q2 = _conv_mm(g2, w2lo, w2hi, b2, sc1lo, sc1hi, sh1lo, sh1hi,
                          apply_affine=True)
    scale2, shift2 = _bn_coeffs(s2, q2, gamma2, beta2)

    return _finalize(h2, x, scale2, shift2)


# speedup vs baseline: 3.0947x; 1.0501x over previous
import functools

import jax
import jax.numpy as jnp
import numpy as np
from jax import lax
from jax.experimental import pallas as pl
from jax.experimental.pallas import tpu as pltpu
from jax.experimental.pallas import tpu_sc as plsc

N = 50000
C = 128
H = 64
K = 7
NK = N * K

NW = 32
NBUF = 4
CHUNK = 112
BPAD = 351232
HALF_B = BPAD // 2
B_PER_W = HALF_B // NW
NCHUNK = B_PER_W // CHUNK
NP = BPAD // K
HALF_NP = NP // 2
MM_R = 896
MM_GRID = HALF_NP // MM_R

_HI_MASK = np.int32(-65536)


def _leaky(v):
    return jnp.where(v > 0, v, 0.2 * v)


def _unpack(w):
    lo = lax.bitcast_convert_type(lax.shift_left(w, 16), jnp.float32)
    hi = lax.bitcast_convert_type(lax.bitwise_and(w, _HI_MASK), jnp.float32)
    return lo, hi


def _pack(a, b):
    ar = a.astype(jnp.bfloat16).astype(jnp.float32)
    br = b.astype(jnp.bfloat16).astype(jnp.float32)
    au = lax.shift_right_logical(lax.bitcast_convert_type(ar, jnp.int32), 16)
    bu = lax.bitwise_and(lax.bitcast_convert_type(br, jnp.int32), _HI_MASK)
    return lax.bitwise_or(au, bu)


def _make_sc_gather(off):
    mesh = plsc.VectorSubcoreMesh(core_axis_name="c", subcore_axis_name="s")

    @functools.partial(
        pl.kernel,
        mesh=mesh,
        compiler_params=pltpu.CompilerParams(use_tc_tiling_on_sc=False),
        out_type=jax.ShapeDtypeStruct((HALF_B, H), jnp.int32),
        scratch_types=[
            pltpu.VMEM((B_PER_W,), jnp.int32),
            pltpu.VMEM((NBUF, CHUNK, H), jnp.int32),
            pltpu.SemaphoreType.DMA,
            pltpu.SemaphoreType.DMA,
        ],
    )
    def gather_kernel(table_hbm, idx_hbm, out_hbm, idx_v, rows_v, semg, semw):
        wid = lax.axis_index("s") * 2 + lax.axis_index("c")
        row0 = wid * B_PER_W
        pltpu.sync_copy(idx_hbm.at[pl.ds(off + row0, B_PER_W)], idx_v)

        def gstart(j):
            buf = rows_v.at[lax.rem(j, NBUF)]
            idx_c = idx_v.at[pl.ds(j * CHUNK, CHUNK)]
            pltpu.make_async_copy(table_hbm.at[idx_c], buf, semg).start()

        def gwait():
            pltpu.make_async_copy(
                table_hbm.at[idx_v.at[pl.ds(0, CHUNK)]], rows_v.at[0], semg
            ).wait()

        def wstart(j):
            pltpu.make_async_copy(
                rows_v.at[lax.rem(j, NBUF)],
                out_hbm.at[pl.ds(row0 + j * CHUNK, CHUNK)],
                semw,
            ).start()

        def wwait():
            pltpu.make_async_copy(
                rows_v.at[0], out_hbm.at[pl.ds(row0, CHUNK)], semw
            ).wait()

        gstart(0)

        def step(j, carry):
            @pl.when(j >= NBUF - 1)
            def _():
                wwait()

            @pl.when(j + 1 < NCHUNK)
            def _():
                gstart(j + 1)

            gwait()
            wstart(j)
            return carry

        lax.fori_loop(0, NCHUNK, step, 0)
        for _ in range(NBUF - 1):
            wwait()

    return gather_kernel


PACK_R = 1000


def _pack_x_body(x_ref, o_ref):
    o_ref[...] = _pack(x_ref[:, :H], x_ref[:, H:])


def _pack_x(x):
    return pl.pallas_call(
        _pack_x_body,
        grid=(N // PACK_R,),
        in_specs=[pl.BlockSpec((PACK_R, C), lambda i: (i, 0))],
        out_specs=pl.BlockSpec((PACK_R, H), lambda i: (i, 0)),
        out_shape=jax.ShapeDtypeStruct((N, H), jnp.int32),
    )(x)


def _mm_body(g_ref, wlo_ref, whi_ref, b_ref, sclo_ref, schi_ref, shlo_ref,
             shhi_ref, h_ref, sum_ref, sq_ref, *, apply_affine, blk_off):
    i = pl.program_id(0)
    glo, ghi = _unpack(g_ref[...])
    if apply_affine:
        glo = _leaky(glo * sclo_ref[...] + shlo_ref[...])
        ghi = _leaky(ghi * schi_ref[...] + shhi_ref[...])
    h = (
        jnp.dot(glo.astype(jnp.bfloat16), wlo_ref[...],
                preferred_element_type=jnp.float32)
        + jnp.dot(ghi.astype(jnp.bfloat16), whi_ref[...],
                  preferred_element_type=jnp.float32)
        + b_ref[...]
    )
    h_ref[...] = _pack(h[:, :H], h[:, H:])
    rows = (i + blk_off) * MM_R + lax.broadcasted_iota(jnp.int32, (MM_R, 1), 0)
    hm = jnp.where(rows < N, h, 0.0)
    s = jnp.sum(hm, axis=0, keepdims=True)
    q = jnp.sum(hm * hm, axis=0, keepdims=True)

    @pl.when(i == 0)
    def _init():
        sum_ref[...] = s
        sq_ref[...] = q

    @pl.when(i > 0)
    def _acc():
        sum_ref[...] += s
        sq_ref[...] += q


def _mm_half_a_call(body, kh):
    return pl.pallas_call(
        body,
        grid=(MM_GRID,),
        in_specs=[
            pl.BlockSpec((MM_R, kh), lambda i: (i, 0)),
            pl.BlockSpec((kh, C), lambda i: (0, 0)),
            pl.BlockSpec((kh, C), lambda i: (0, 0)),
            pl.BlockSpec((1, C), lambda i: (0, 0)),
            pl.BlockSpec((1, kh), lambda i: (0, 0)),
            pl.BlockSpec((1, kh), lambda i: (0, 0)),
            pl.BlockSpec((1, kh), lambda i: (0, 0)),
            pl.BlockSpec((1, kh), lambda i: (0, 0)),
        ],
        out_specs=[
            pl.BlockSpec((MM_R, H), lambda i: (i, 0)),
            pl.BlockSpec((1, C), lambda i: (0, 0)),
            pl.BlockSpec((1, C), lambda i: (0, 0)),
        ],
        out_shape=[
            jax.ShapeDtypeStruct((NP, H), jnp.int32),
            jax.ShapeDtypeStruct((1, C), jnp.float32),
            jax.ShapeDtypeStruct((1, C), jnp.float32),
        ],
    )


def _mm_half_b_call(body, kh):
    nb = MM_GRID
    return pl.pallas_call(
        body,
        grid=(MM_GRID,),
        in_specs=[
            pl.BlockSpec((MM_R, kh), lambda i: (i, 0)),
            pl.BlockSpec((kh, C), lambda i: (0, 0)),
            pl.BlockSpec((kh, C), lambda i: (0, 0)),
            pl.BlockSpec((1, C), lambda i: (0, 0)),
            pl.BlockSpec((1, kh), lambda i: (0, 0)),
            pl.BlockSpec((1, kh), lambda i: (0, 0)),
            pl.BlockSpec((1, kh), lambda i: (0, 0)),
            pl.BlockSpec((1, kh), lambda i: (0, 0)),
            pl.BlockSpec(memory_space=pl.ANY),
        ],
        out_specs=[
            pl.BlockSpec((MM_R, H), lambda i: (i + nb, 0)),
            pl.BlockSpec((1, C), lambda i: (0, 0)),
            pl.BlockSpec((1, C), lambda i: (0, 0)),
        ],
        out_shape=[
            jax.ShapeDtypeStruct((NP, H), jnp.int32),
            jax.ShapeDtypeStruct((1, C), jnp.float32),
            jax.ShapeDtypeStruct((1, C), jnp.float32),
        ],
        input_output_aliases={8: 0},
    )


def _conv(ga, gb, wlo, whi, b, sclo, schi, shlo, shhi, apply_affine):
    kh = K * H
    body_a = functools.partial(_mm_body, apply_affine=apply_affine, blk_off=0)
    body_b = functools.partial(_mm_body, apply_affine=apply_affine,
                               blk_off=MM_GRID)

    def ignore_alias(*refs):
        body_b(*refs[:8], *refs[9:])

    args = (wlo, whi, b.reshape(1, C), sclo, schi, shlo, shhi)
    ha, sa, qa = _mm_half_a_call(body_a, kh)(ga.reshape(HALF_NP, kh), *args)
    hb, sb, qb = _mm_half_b_call(ignore_alias, kh)(
        gb.reshape(HALF_NP, kh), *args, ha)
    return hb, sa + sb, qa + qb


FIN_R = 1000


def _fin_body(h_ref, x_ref, sc_ref, sh_ref, o_ref):
    hlo, hhi = _unpack(h_ref[...])
    h = jnp.concatenate([hlo, hhi], axis=1)
    o_ref[...] = _leaky(h * sc_ref[...] + sh_ref[...] + x_ref[...])


def _finalize(h2, x, scale2, shift2):
    return pl.pallas_call(
        _fin_body,
        grid=(N // FIN_R,),
        in_specs=[
            pl.BlockSpec((FIN_R, H), lambda i: (i, 0)),
            pl.BlockSpec((FIN_R, C), lambda i: (i, 0)),
            pl.BlockSpec((1, C), lambda i: (0, 0)),
            pl.BlockSpec((1, C), lambda i: (0, 0)),
        ],
        out_specs=pl.BlockSpec((FIN_R, C), lambda i: (i, 0)),
        out_shape=jax.ShapeDtypeStruct((N, C), jnp.float32),
    )(h2, x, scale2.reshape(1, C), shift2.reshape(1, C))


def _bn_coeffs(ssum, ssq, gamma, beta):
    mean = ssum[0] / N
    var = ssq[0] / N - mean * mean
    scale = gamma * lax.rsqrt(var + 1e-5)
    shift = beta - mean * scale
    return scale, shift


def _split_w(w):
    w3 = w.reshape(K, C, C)
    wlo = w3[:, :H, :].reshape(K * H, C)
    whi = w3[:, H:, :].reshape(K * H, C)
    return wlo, whi


def kernel(x, neigh_orders, W1, b1, gamma1, beta1, W2, b2, gamma2, beta2):
    idx = jnp.concatenate(
        [neigh_orders, jnp.zeros((BPAD - NK,), jnp.int32)]
    )
    kh = K * H
    zeros_kh = jnp.zeros((1, kh), jnp.float32)
    w1lo, w1hi = _split_w(W1.astype(jnp.bfloat16))
    w2lo, w2hi = _split_w(W2.astype(jnp.bfloat16))

    gather_a = _make_sc_gather(0)
    gather_b = _make_sc_gather(HALF_B)

    xp = _pack_x(x)
    g1a = gather_a(xp, idx)
    g1b = gather_b(xp, idx)
    h1, s1, q1 = _conv(g1a, g1b, w1lo, w1hi, b1, zeros_kh, zeros_kh,
                       zeros_kh, zeros_kh, apply_affine=False)
    scale1, shift1 = _bn_coeffs(s1, q1, gamma1, beta1)

    g2a = gather_a(h1, idx)
    g2b = gather_b(h1, idx)
    sc1lo = jnp.tile(scale1[:H], K).reshape(1, kh)
    sc1hi = jnp.tile(scale1[H:], K).reshape(1, kh)
    sh1lo = jnp.tile(shift1[:H], K).reshape(1, kh)
    sh1hi = jnp.tile(shift1[H:], K).reshape(1, kh)
    h2, s2, q2 = _conv(g2a, g2b, w2lo, w2hi, b2, sc1lo, sc1hi, sh1lo, sh1hi,
                       apply_affine=True)
    scale2, shift2 = _bn_coeffs(s2, q2, gamma2, beta2)

    return _finalize(h2, x, scale2, shift2)
